# baseline (speedup 1.0000x reference)
import functools

import jax
import jax.numpy as jnp
from jax import lax
from jax.experimental import pallas as pl
from jax.experimental.pallas import tpu as pltpu
from jax.experimental.pallas import tpu_sc as plsc

BB = 4
K5 = 5


def _make_topcand_kernel(G, B, S):
    info = plsc.get_sparse_core_info()
    NC, NS, L = info.num_cores, info.num_subcores, info.num_lanes
    NW = NC * NS
    BPW = B // NW
    mesh = plsc.VectorSubcoreMesh(core_axis_name="c", subcore_axis_name="s")

    @functools.partial(
        pl.kernel,
        mesh=mesh,
        out_type=jax.ShapeDtypeStruct((NW, G, BPW, K5, L), jnp.float32),
        scratch_types=[
            pltpu.VMEM((G, BPW, S), jnp.float32),
            pltpu.VMEM((G, BPW, K5, L), jnp.float32),
        ],
    )
    def topcand_kernel(sim_hbm, out_hbm, rows_v, tops_v):
        wid = lax.axis_index("s") * NC + lax.axis_index("c")
        b0 = wid * BPW
        for g in range(G):
            pltpu.sync_copy(sim_hbm.at[g, pl.ds(b0, BPW)], rows_v.at[g])
        for g in range(G):
            for bl in range(BPW):
                row = rows_v.at[g, bl]
                neg = jnp.full((L,), -jnp.inf, jnp.float32)

                def chunk(i, t, row=row):
                    t1, t2, t3, t4, t5 = t
                    for u in range(4):
                        v = row[pl.ds((i * 4 + u) * L, L)]
                        m = jnp.maximum(t1, v); v = jnp.minimum(t1, v); t1 = m
                        m = jnp.maximum(t2, v); v = jnp.minimum(t2, v); t2 = m
                        m = jnp.maximum(t3, v); v = jnp.minimum(t3, v); t3 = m
                        m = jnp.maximum(t4, v); v = jnp.minimum(t4, v); t4 = m
                        m = jnp.maximum(t5, v); v = jnp.minimum(t5, v); t5 = m
                    return (t1, t2, t3, t4, t5)

                tops = lax.fori_loop(
                    0, S // (4 * L), chunk, (neg, neg, neg, neg, neg))
                for j in range(K5):
                    tops_v[g, bl, j] = tops[j]
        pltpu.sync_copy(tops_v, out_hbm.at[wid])

    return topcand_kernel, NW, L


def _fuse_body(cand_ref, base_ref, p_ref, r_ref, out_ref, alpha_ref):
    x = cand_ref[0]
    G, Bb, C = x.shape
    iota = lax.broadcasted_iota(jnp.int32, (G, Bb, C), 2)
    acc = jnp.zeros((G, Bb, 1), jnp.float32)
    for i in range(K5):
        m = jnp.max(x, axis=-1, keepdims=True)
        acc = acc + m
        if i < K5 - 1:
            eq = x == m
            first = jnp.min(jnp.where(eq, iota, C), axis=-1, keepdims=True)
            x = jnp.where(iota == first, -jnp.inf, x)
    conf = jnp.sum(acc, axis=(0, 2)) / (5.0 * G)
    z = base_ref[0, 0] - conf * 10.0
    a = 1.0 / (1.0 + jnp.exp(-z))
    a = jnp.clip(a, 0.1, 0.9)
    b = pl.program_id(0)
    alpha_ref[pl.ds(b * Bb, Bb)] = a.reshape(Bb, 1, 1)
    a3 = a.reshape(Bb, 1, 1)
    out_ref[...] = a3 * p_ref[...] + (1.0 - a3) * r_ref[...]


def kernel(param_pred, retrieval_pred, retrieval_sim, base_alpha):
    B, T, D = param_pred.shape
    G, _, S = retrieval_sim.shape
    topcand, NW, L = _make_topcand_kernel(G, B, S)
    cand = topcand(retrieval_sim)
    cand4 = cand.reshape(NW, G, B // NW, K5 * L)
    base = jnp.reshape(base_alpha, (1, 1)).astype(jnp.float32)

    fused, alpha = pl.pallas_call(
        _fuse_body,
        grid=(B // BB,),
        in_specs=[
            pl.BlockSpec((1, G, BB, K5 * L), lambda b: (b, 0, 0, 0)),
            pl.BlockSpec((1, 1), lambda b: (0, 0)),
            pl.BlockSpec((BB, T, D), lambda b: (b, 0, 0)),
            pl.BlockSpec((BB, T, D), lambda b: (b, 0, 0)),
        ],
        out_specs=[
            pl.BlockSpec((BB, T, D), lambda b: (b, 0, 0)),
            pl.BlockSpec((B, 1, 1), lambda b: (0, 0, 0)),
        ],
        out_shape=[
            jax.ShapeDtypeStruct((B, T, D), jnp.float32),
            jax.ShapeDtypeStruct((B, 1, 1), jnp.float32),
        ],
        compiler_params=pltpu.CompilerParams(
            dimension_semantics=("arbitrary",),
        ),
    )(cand4, base, param_pred, retrieval_pred)
    return fused, alpha.reshape(B)
